# baseline (speedup 1.0000x reference)
import functools

import jax
import jax.numpy as jnp
from jax import lax
from jax.experimental import pallas as pl
from jax.experimental.pallas import tpu as pltpu
from jax.experimental.pallas import tpu_sc as plsc

_N = 10000
_E = 160000
_L = 64
_W = 128
_STEPS = 6
_NW = 32
_CHUNK = 128
_CPW = 40
_EPW = _CHUNK * _CPW
_EP = _NW * _EPW
_NP = 10112
_RPT = _NP // 16
_BE = 2048


def _mesh():
    return plsc.VectorSubcoreMesh(core_axis_name="c", subcore_axis_name="s")


def _widen(v, rows):
    return jnp.concatenate([v, jnp.zeros((rows, _W - v.shape[1]), jnp.float32)],
                           axis=1)


def _split_bf16(v):
    hi = v.astype(jnp.bfloat16)
    lo = (v - hi.astype(jnp.float32)).astype(jnp.bfloat16)
    return jnp.concatenate([hi, lo], axis=1)


def _join_bf16(hl):
    return hl[:, :_L].astype(jnp.float32) + hl[:, _L:].astype(jnp.float32)


_NB = 4


def _sc_gather(table, idx):
    s_sets = idx.shape[0]
    t_total = s_sets * _CPW

    @functools.partial(
        pl.kernel,
        out_type=jax.ShapeDtypeStruct((s_sets * _EP, _W), jnp.float32),
        mesh=_mesh(),
        scratch_types=(
            [pltpu.VMEM((s_sets * _CPW, _CHUNK), jnp.int32)]
            + [pltpu.VMEM((_CHUNK, _W), jnp.float32)] * _NB
            + [pltpu.SemaphoreType.DMA] * (2 * _NB)
        ),
    )
    def k(table_hbm, idx_hbm, out_hbm, idx_v, *rest):
        bufs = rest[:_NB]
        gs = rest[_NB:2 * _NB]
        os_ = rest[2 * _NB:3 * _NB]
        cid = lax.axis_index("c")
        sid = lax.axis_index("s")
        w = cid * 16 + sid
        base = w * _EPW
        for a in range(s_sets):
            pltpu.sync_copy(idx_hbm.at[a, w],
                            idx_v.at[pl.ds(a * _CPW, _CPW)])

        def out_off(t):
            if s_sets == 1:
                return base + t * _CHUNK
            a = (t >= _CPW).astype(jnp.int32)
            return a * _EP + base + (t - a * _CPW) * _CHUNK

        def start_gather(t, buf, sem):
            pltpu.async_copy(table_hbm.at[idx_v.at[t]], buf, sem)

        def wait_gather(buf, sem):
            pltpu.make_async_copy(table_hbm.at[idx_v.at[0]], buf,
                                  sem).wait()

        def start_out(t, buf, sem):
            pltpu.async_copy(buf, out_hbm.at[pl.ds(out_off(t), _CHUNK)],
                             sem)

        def wait_out(buf, sem):
            pltpu.make_async_copy(buf, out_hbm.at[pl.ds(0, _CHUNK)],
                                  sem).wait()

        for b in range(_NB):
            start_gather(b, bufs[b], gs[b])

        half = _NB // 2

        @pl.loop(0, t_total // _NB)
        def _(u):
            tb = _NB * u
            for b in range(_NB):
                t = tb + b
                wait_gather(bufs[b], gs[b])
                start_out(t, bufs[b], os_[b])
                bp = (b + half) % _NB

                @pl.when(jnp.logical_and(t >= half, t + half < t_total))
                def _():
                    wait_out(bufs[bp], os_[bp])
                    start_gather(t + half, bufs[bp], gs[bp])

        for b in range(_NB):
            wait_out(bufs[b], os_[b])

    return k(table, idx)


def _sc_scatter_add(data, idx, zrows):
    nbs = 2

    @functools.partial(
        pl.kernel,
        out_type=jax.ShapeDtypeStruct((2, _NP, _W), jnp.float32),
        mesh=_mesh(),
        scratch_types=(
            [pltpu.VMEM((_CPW, _CHUNK), jnp.int32)]
            + [pltpu.VMEM((_CHUNK, _W), jnp.float32)] * nbs
            + [pltpu.SemaphoreType.DMA] * (2 * nbs)
            + [pltpu.VMEM_SHARED((_NP, _W), jnp.float32)]
        ),
    )
    def k(data_hbm, idx_hbm, z_hbm, out_hbm, idx_v, *rest):
        bufs = rest[:nbs]
        ls = rest[nbs:2 * nbs]
        ss = rest[2 * nbs:3 * nbs]
        acc = rest[3 * nbs]
        cid = lax.axis_index("c")
        sid = lax.axis_index("s")
        w = cid * 16 + sid
        base = w * _EPW
        pltpu.sync_copy(z_hbm, acc.at[pl.ds(sid * _RPT, _RPT)])
        pltpu.sync_copy(idx_hbm.at[w], idx_v)
        plsc.subcore_barrier()

        def start_load(t, buf, sem):
            pltpu.async_copy(data_hbm.at[pl.ds(base + t * _CHUNK, _CHUNK)],
                             buf, sem)

        def wait_load(buf, sem):
            pltpu.make_async_copy(data_hbm.at[pl.ds(0, _CHUNK)], buf,
                                  sem).wait()

        def start_scat(t, buf, sem):
            pltpu.async_copy(buf, acc.at[idx_v.at[t]], sem, add=True)

        def wait_scat(buf, sem):
            pltpu.make_async_copy(buf, acc.at[idx_v.at[0]], sem).wait()

        for b in range(nbs):
            start_load(b, bufs[b], ls[b])

        @pl.loop(0, _CPW // nbs)
        def _(u):
            tb = nbs * u
            for b in range(nbs):
                t = tb + b
                wait_load(bufs[b], ls[b])
                start_scat(t, bufs[b], ss[b])

                @pl.when(t + nbs < _CPW)
                def _():
                    wait_scat(bufs[b], ss[b])
                    start_load(t + nbs, bufs[b], ls[b])

        for b in range(nbs):
            wait_scat(bufs[b], ss[b])
        plsc.subcore_barrier()
        pltpu.sync_copy(acc.at[pl.ds(sid * _RPT, _RPT)],
                        out_hbm.at[cid, pl.ds(sid * _RPT, _RPT)])

    return k(data, idx, zrows)


def _full(shape):
    return pl.BlockSpec(shape, lambda i: tuple(0 for _ in shape))


def _tc_edge_enc(ea, w0, b0, w1, b1):
    def body(ea_ref, w0_ref, b0_ref, w1_ref, b1_ref, out_ref):
        h = jnp.maximum(ea_ref[...] @ w0_ref[...] + b0_ref[...], 0.0)
        out_ref[...] = h @ w1_ref[...] + b1_ref[...]

    return pl.pallas_call(
        body,
        grid=(_EP // _BE,),
        in_specs=[
            pl.BlockSpec((_BE, 8), lambda i: (i, 0)),
            _full((8, _L)), _full((1, _L)), _full((_L, _L)), _full((1, _L)),
        ],
        out_specs=pl.BlockSpec((_BE, _L), lambda i: (i, 0)),
        out_shape=jax.ShapeDtypeStruct((_EP, _L), jnp.float32),
    )(ea, w0, b0, w1, b1)


def _tc_prep(x, cnt_a, cnt_b, gf, a0, ab0, a1, ab1, g0, gb0, g1, gb1,
             wg_e, bg_e, wg_n, bg_n):
    def body(x_ref, ca_ref, cb_ref, gf_ref, a0_ref, ab0_ref, a1_ref, ab1_ref,
             g0_ref, gb0_ref, g1_ref, gb1_ref, wge_ref, bge_ref, wgn_ref,
             bgn_ref, nl_ref, inv_ref, g_ref, egt_ref, ngt_ref):
        h = jnp.maximum(x_ref[...] @ a0_ref[...] + ab0_ref[...], 0.0)
        nl_ref[...] = _widen(h @ a1_ref[...] + ab1_ref[...], _NP)
        inv_ref[...] = 1.0 / jnp.maximum(ca_ref[...] + cb_ref[...], 1.0)
        hg = jnp.maximum(gf_ref[...] @ g0_ref[...] + gb0_ref[...], 0.0)
        g = hg @ g1_ref[...] + gb1_ref[...]
        g_ref[...] = g
        egt_ref[...] = g @ wge_ref[...] + bge_ref[...]
        ngt_ref[...] = g @ wgn_ref[...] + bgn_ref[...]

    return pl.pallas_call(
        body,
        out_shape=[
            jax.ShapeDtypeStruct((_NP, _W), jnp.float32),
            jax.ShapeDtypeStruct((_NP, _W), jnp.float32),
            jax.ShapeDtypeStruct((1, _L), jnp.float32),
            jax.ShapeDtypeStruct((1, _L), jnp.float32),
            jax.ShapeDtypeStruct((1, _L), jnp.float32),
        ],
    )(x, cnt_a, cnt_b, gf, a0, ab0, a1, ab1, g0, gb0, g1, gb1,
      wg_e, bg_e, wg_n, bg_n)


def _tc_edge(nlr, nlc, el, es, ws, bs, w0, egt, w1, b1):
    def body(nlr_ref, nlc_ref, el_ref, es_ref, ws_ref, bs_ref, w0_ref,
             egt_ref, w1_ref, b1_ref, out_ref):
        ein = (jnp.concatenate([el_ref[:, :_L], es_ref[...]], axis=1)
               @ ws_ref[...] + bs_ref[...])
        h = jnp.maximum(
            jnp.concatenate([nlr_ref[:, :_L], nlc_ref[:, :_L], ein], axis=1)
            @ w0_ref[...] + egt_ref[...], 0.0)
        out_ref[...] = _widen(h @ w1_ref[...] + b1_ref[...], _BE)

    ebw = pl.BlockSpec((_BE, _W), lambda i: (i, 0))
    ebl = pl.BlockSpec((_BE, _L), lambda i: (i, 0))
    return pl.pallas_call(
        body,
        grid=(_EP // _BE,),
        in_specs=[ebw, ebw, ebw, ebl,
                  _full((2 * _L, _L)), _full((1, _L)),
                  _full((3 * _L, _L)), _full((1, _L)),
                  _full((_L, _L)), _full((1, _L))],
        out_specs=ebw,
        out_shape=jax.ShapeDtypeStruct((_EP, _W), jnp.float32),
    )(nlr, nlc, el, es, ws, bs, w0, egt, w1, b1)


def _tc_node(nl, s_a, s_b, inv, v0, ngt, v1, b1, g0, gb0, g1, gb1, g,
             wg_e, bg_e, wg_n, bg_n):
    def body(nl_ref, sa_ref, sb_ref, inv_ref, v0_ref, ngt_ref, v1_ref,
             b1_ref, g0_ref, gb0_ref, g1_ref, gb1_ref, g_ref, wge_ref,
             bge_ref, wgn_ref, bgn_ref, ne_ref, gn_ref, egt_ref, ngt2_ref):
        s = sa_ref[:, :_L] + sb_ref[:, :_L]
        agg = s * inv_ref[:, :_L]
        h = jnp.maximum(
            jnp.concatenate([nl_ref[:, :_L], agg], axis=1) @ v0_ref[...]
            + ngt_ref[...], 0.0)
        ne = h @ v1_ref[...] + b1_ref[...]
        ne_ref[...] = _widen(ne, _NP)
        ridx = lax.broadcasted_iota(jnp.int32, (_NP, 1), 0)
        valid = (ridx < _N).astype(jnp.float32)
        n_g = jnp.sum(ne * valid, axis=0, keepdims=True) * (1.0 / _N)
        e_g = jnp.sum(s * valid, axis=0, keepdims=True) * (1.0 / _E)
        hg = jnp.maximum(
            jnp.concatenate([n_g, e_g, g_ref[...]], axis=1) @ g0_ref[...]
            + gb0_ref[...], 0.0)
        gn = hg @ g1_ref[...] + gb1_ref[...]
        gn_ref[...] = gn
        egt_ref[...] = gn @ wge_ref[...] + bge_ref[...]
        ngt2_ref[...] = gn @ wgn_ref[...] + bgn_ref[...]

    return pl.pallas_call(
        body,
        out_shape=[
            jax.ShapeDtypeStruct((_NP, _W), jnp.float32),
            jax.ShapeDtypeStruct((1, _L), jnp.float32),
            jax.ShapeDtypeStruct((1, _L), jnp.float32),
            jax.ShapeDtypeStruct((1, _L), jnp.float32),
        ],
    )(nl, s_a, s_b, inv, v0, ngt, v1, b1, g0, gb0, g1, gb1, g,
      wg_e, bg_e, wg_n, bg_n)


def _tc_diag(nl, d0, db0, d1, db1):
    def body(nl_ref, d0_ref, db0_ref, d1_ref, db1_ref, out_ref):
        h = jnp.maximum(nl_ref[:, :_L] @ d0_ref[...] + db0_ref[...], 0.0)
        dc = h @ d1_ref[...] + db1_ref[...]
        out_ref[...] = jnp.concatenate(
            [dc, jnp.zeros((_NP, _W - 2), jnp.float32)], axis=1)

    return pl.pallas_call(
        body,
        out_shape=jax.ShapeDtypeStruct((_NP, _W), jnp.float32),
    )(nl, d0, db0, d1, db1)


def _tc_final(el, dcr, row3, col3, e0, eb0, e1, eb1):
    nb = _EP // _BE

    def body(el_ref, dcr_ref, row_ref, col_ref, e0_ref, eb0_ref, e1_ref,
             eb1_ref, m_ref, l1_ref):
        i = pl.program_id(0)

        @pl.when(i == 0)
        def _():
            l1_ref[...] = jnp.zeros((1, 2), jnp.float32)

        h = jnp.maximum(el_ref[:, :_L] @ e0_ref[...] + eb0_ref[...], 0.0)
        ec = h @ e1_ref[...] + eb1_ref[...]
        r = row_ref[0]
        c = col_ref[0]
        diag = r == c
        dre = dcr_ref[:, 0:1]
        dim = dcr_ref[:, 1:2]
        m_re = jnp.where(diag, 1.0 + dre, ec[:, 0:1])
        m_im = jnp.where(diag, dim, ec[:, 1:2])
        m_ref[...] = jnp.concatenate([m_re, m_im], axis=1)
        eidx = i * _BE + lax.broadcasted_iota(jnp.int32, (_BE, 1), 0)
        offv = jnp.logical_and(jnp.logical_not(diag), eidx < _E)
        sq = m_re * m_re + m_im * m_im
        mag = jnp.sqrt(jnp.where(offv, sq, 1.0))
        bsum = jnp.sum(jnp.where(offv, mag, 0.0))
        bcnt = jnp.sum(offv.astype(jnp.float32))
        upd = jnp.concatenate(
            [jnp.full((1, 1), bsum), jnp.full((1, 1), bcnt)], axis=1)
        l1_ref[...] = l1_ref[...] + upd

        @pl.when(i == nb - 1)
        def _():
            v = l1_ref[...]
            l1 = v[:, 0:1] / jnp.maximum(v[:, 1:2], 1.0)
            l1_ref[...] = jnp.concatenate([l1, v[:, 1:2]], axis=1)

    ebw = pl.BlockSpec((_BE, _W), lambda i: (i, 0))
    ib = pl.BlockSpec((1, _BE, 1), lambda i: (i, 0, 0))
    return pl.pallas_call(
        body,
        grid=(nb,),
        in_specs=[ebw, ebw, ib, ib,
                  _full((_L, _L)), _full((1, _L)),
                  _full((_L, 2)), _full((1, 2))],
        out_specs=[pl.BlockSpec((_BE, 2), lambda i: (i, 0)),
                   pl.BlockSpec((1, 2), lambda i: (0, 0))],
        out_shape=[jax.ShapeDtypeStruct((_EP, 2), jnp.float32),
                   jax.ShapeDtypeStruct((1, 2), jnp.float32)],
    )(el, dcr, row3, col3, e0, eb0, e1, eb1)


def kernel(x, edge_attr, global_features, params, edge_index):
    f32 = jnp.float32
    row = edge_index[0].astype(jnp.int32)
    col = edge_index[1].astype(jnp.int32)
    pad_e = _EP - _E

    row_g = jnp.concatenate([row, jnp.zeros((pad_e,), jnp.int32)])
    col_g = jnp.concatenate([col, jnp.zeros((pad_e,), jnp.int32)])
    idx_g2 = jnp.stack([row_g, col_g]).reshape(2, _NW, _CPW, _CHUNK)
    idx_g1 = row_g.reshape(1, _NW, _CPW, _CHUNK)
    idx_s = jnp.concatenate(
        [row, jnp.full((pad_e,), _N, jnp.int32)]).reshape(
            _NW, _CPW, _CHUNK)
    row3 = row_g.reshape(_EP // _BE, _BE, 1)
    col3 = col_g.reshape(_EP // _BE, _BE, 1)

    zrows = jnp.zeros((_RPT, _W), f32)
    ones_d = jnp.ones((_EP, _W), f32)
    x_pad = jnp.concatenate([x, jnp.zeros((_NP - _N, 9), f32)])
    ea_pad = jnp.concatenate([edge_attr, jnp.zeros((pad_e, 8), f32)])
    gf = global_features.reshape(1, 4)

    p = params

    def wb(mlp, i):
        return mlp[i]["W"], mlp[i]["b"].reshape(1, -1)

    cnt = _sc_scatter_add(ones_d, idx_s, zrows)

    b0w, b0b = wb(p["edge_enc"], 0)
    b1w, b1b = wb(p["edge_enc"], 1)
    es = _tc_edge_enc(ea_pad, b0w, b0b, b1w, b1b)

    a0w, a0b = wb(p["node_enc"], 0)
    a1w, a1b = wb(p["node_enc"], 1)
    g0w, g0b = wb(p["global_enc"], 0)
    g1w, g1b = wb(p["global_enc"], 1)
    e0 = p["proc"][0]
    nl, inv, g, egt, ngt = _tc_prep(
        x_pad, cnt[0], cnt[1], gf, a0w, a0b, a1w, a1b, g0w, g0b, g1w, g1b,
        e0["edge"][0]["W"][0:_L], e0["edge"][0]["b"].reshape(1, -1),
        e0["node"][0]["W"][0:_L], e0["node"][0]["b"].reshape(1, -1))

    el = _widen(es, _EP)
    for i in range(_STEPS):
        blk = p["proc"][i]
        sp = p["skip"][i]
        nlrc = _sc_gather(nl, idx_g2)
        ew0 = blk["edge"][0]["W"][_L:4 * _L]
        ew1, eb1 = wb(blk["edge"], 1)
        e_emb = _tc_edge(nlrc[:_EP], nlrc[_EP:], el, es,
                         sp["W"], sp["b"].reshape(1, -1), ew0, egt, ew1, eb1)
        sums = _sc_scatter_add(e_emb, idx_s, zrows)
        nxt = p["proc"][(i + 1) % _STEPS]
        nw0 = blk["node"][0]["W"][_L:3 * _L]
        nw1, nb1 = wb(blk["node"], 1)
        gw0, gb0 = wb(blk["global"], 0)
        gw1, gb1 = wb(blk["global"], 1)
        nl, g, egt, ngt = _tc_node(
            nl, sums[0], sums[1], inv, nw0, ngt, nw1, nb1,
            gw0, gb0, gw1, gb1, g,
            nxt["edge"][0]["W"][0:_L], nxt["edge"][0]["b"].reshape(1, -1),
            nxt["node"][0]["W"][0:_L], nxt["node"][0]["b"].reshape(1, -1))
        el = e_emb

    d0w, d0b = wb(p["diag_dec"], 0)
    d1w, d1b = wb(p["diag_dec"], 1)
    dpad = _tc_diag(nl, d0w, d0b, d1w, d1b)
    dcr = _sc_gather(dpad, idx_g1)

    ed0w, ed0b = wb(p["edge_dec"], 0)
    ed1w, ed1b = wb(p["edge_dec"], 1)
    m, l1v = _tc_final(el, dcr, row3, col3, ed0w, ed0b, ed1w, ed1b)
    return m[:_E], l1v[0, 0]


# speedup vs baseline: 1.7180x; 1.7180x over previous
import functools

import jax
import jax.numpy as jnp
from jax import lax
from jax.experimental import pallas as pl
from jax.experimental.pallas import tpu as pltpu
from jax.experimental.pallas import tpu_sc as plsc

_N = 10000
_E = 160000
_L = 64
_W = 128
_STEPS = 6
_NW = 32
_CHUNK = 128
_CPW = 40
_EPW = _CHUNK * _CPW
_EP = _NW * _EPW
_NP = 10112
_RPT = _NP // 16
_BE = 2048


def _mesh():
    return plsc.VectorSubcoreMesh(core_axis_name="c", subcore_axis_name="s")


def _widen(v, rows):
    return jnp.concatenate([v, jnp.zeros((rows, _W - v.shape[1]), jnp.float32)],
                           axis=1)


def _split_bf16(v):
    hi = v.astype(jnp.bfloat16)
    lo = (v - hi.astype(jnp.float32)).astype(jnp.bfloat16)
    return jnp.concatenate([hi, lo], axis=1)


def _join_bf16(hl):
    return hl[:, :_L].astype(jnp.float32) + hl[:, _L:].astype(jnp.float32)


_NB = 4
_NG = 2


def _sc_gather(table, idx):
    s_sets = idx.shape[0]
    t_total = s_sets * _CPW

    @functools.partial(
        pl.kernel,
        out_type=jax.ShapeDtypeStruct((s_sets * _EP, _W), jnp.float32),
        mesh=_mesh(),
        scratch_types=(
            [pltpu.VMEM((s_sets * _CPW, _CHUNK), jnp.int32)]
            + [pltpu.VMEM((_CHUNK, _W), jnp.float32)] * _NG
            + [pltpu.SemaphoreType.DMA] * (2 * _NG)
            + [pltpu.VMEM_SHARED((_NP, _W), jnp.float32)]
        ),
    )
    def k(table_hbm, idx_hbm, out_hbm, idx_v, *rest):
        bufs = rest[:_NG]
        gs = rest[_NG:2 * _NG]
        os_ = rest[2 * _NG:3 * _NG]
        tbl = rest[3 * _NG]
        cid = lax.axis_index("c")
        sid = lax.axis_index("s")
        w = cid * 16 + sid
        base = w * _EPW
        pltpu.sync_copy(table_hbm.at[pl.ds(sid * _RPT, _RPT)],
                        tbl.at[pl.ds(sid * _RPT, _RPT)])
        for a in range(s_sets):
            pltpu.sync_copy(idx_hbm.at[a, w],
                            idx_v.at[pl.ds(a * _CPW, _CPW)])
        plsc.subcore_barrier()

        def out_off(t):
            if s_sets == 1:
                return base + t * _CHUNK
            a = (t >= _CPW).astype(jnp.int32)
            return a * _EP + base + (t - a * _CPW) * _CHUNK

        def start_gather(t, buf, sem):
            pltpu.async_copy(tbl.at[idx_v.at[t]], buf, sem)

        def wait_gather(buf, sem):
            pltpu.make_async_copy(tbl.at[idx_v.at[0]], buf, sem).wait()

        def start_out(t, buf, sem):
            pltpu.async_copy(buf, out_hbm.at[pl.ds(out_off(t), _CHUNK)],
                             sem)

        def wait_out(buf, sem):
            pltpu.make_async_copy(buf, out_hbm.at[pl.ds(0, _CHUNK)],
                                  sem).wait()

        for b in range(_NG):
            start_gather(b, bufs[b], gs[b])

        half = _NG // 2

        @pl.loop(0, t_total // _NG)
        def _(u):
            tb = _NG * u
            for b in range(_NG):
                t = tb + b
                wait_gather(bufs[b], gs[b])
                start_out(t, bufs[b], os_[b])
                bp = (b + half) % _NG

                @pl.when(jnp.logical_and(t >= half, t + half < t_total))
                def _():
                    wait_out(bufs[bp], os_[bp])
                    start_gather(t + half, bufs[bp], gs[bp])

        for b in range(_NG):
            wait_out(bufs[b], os_[b])

    return k(table, idx)


def _sc_scatter_add(data, idx, zrows):
    nbs = 2

    @functools.partial(
        pl.kernel,
        out_type=jax.ShapeDtypeStruct((2, _NP, _W), jnp.float32),
        mesh=_mesh(),
        scratch_types=(
            [pltpu.VMEM((_CPW, _CHUNK), jnp.int32)]
            + [pltpu.VMEM((_CHUNK, _W), jnp.float32)] * nbs
            + [pltpu.SemaphoreType.DMA] * (2 * nbs)
            + [pltpu.VMEM_SHARED((_NP, _W), jnp.float32)]
        ),
    )
    def k(data_hbm, idx_hbm, z_hbm, out_hbm, idx_v, *rest):
        bufs = rest[:nbs]
        ls = rest[nbs:2 * nbs]
        ss = rest[2 * nbs:3 * nbs]
        acc = rest[3 * nbs]
        cid = lax.axis_index("c")
        sid = lax.axis_index("s")
        w = cid * 16 + sid
        base = w * _EPW
        pltpu.sync_copy(z_hbm, acc.at[pl.ds(sid * _RPT, _RPT)])
        pltpu.sync_copy(idx_hbm.at[w], idx_v)
        plsc.subcore_barrier()

        def start_load(t, buf, sem):
            pltpu.async_copy(data_hbm.at[pl.ds(base + t * _CHUNK, _CHUNK)],
                             buf, sem)

        def wait_load(buf, sem):
            pltpu.make_async_copy(data_hbm.at[pl.ds(0, _CHUNK)], buf,
                                  sem).wait()

        def start_scat(t, buf, sem):
            pltpu.async_copy(buf, acc.at[idx_v.at[t]], sem, add=True)

        def wait_scat(buf, sem):
            pltpu.make_async_copy(buf, acc.at[idx_v.at[0]], sem).wait()

        for b in range(nbs):
            start_load(b, bufs[b], ls[b])

        @pl.loop(0, _CPW // nbs)
        def _(u):
            tb = nbs * u
            for b in range(nbs):
                t = tb + b
                wait_load(bufs[b], ls[b])
                start_scat(t, bufs[b], ss[b])

                @pl.when(t + nbs < _CPW)
                def _():
                    wait_scat(bufs[b], ss[b])
                    start_load(t + nbs, bufs[b], ls[b])

        for b in range(nbs):
            wait_scat(bufs[b], ss[b])
        plsc.subcore_barrier()
        pltpu.sync_copy(acc.at[pl.ds(sid * _RPT, _RPT)],
                        out_hbm.at[cid, pl.ds(sid * _RPT, _RPT)])

    return k(data, idx, zrows)


def _full(shape):
    return pl.BlockSpec(shape, lambda i: tuple(0 for _ in shape))


def _tc_edge_enc(ea, w0, b0, w1, b1):
    def body(ea_ref, w0_ref, b0_ref, w1_ref, b1_ref, out_ref):
        h = jnp.maximum(ea_ref[...] @ w0_ref[...] + b0_ref[...], 0.0)
        out_ref[...] = h @ w1_ref[...] + b1_ref[...]

    return pl.pallas_call(
        body,
        grid=(_EP // _BE,),
        in_specs=[
            pl.BlockSpec((_BE, 8), lambda i: (i, 0)),
            _full((8, _L)), _full((1, _L)), _full((_L, _L)), _full((1, _L)),
        ],
        out_specs=pl.BlockSpec((_BE, _L), lambda i: (i, 0)),
        out_shape=jax.ShapeDtypeStruct((_EP, _L), jnp.float32),
    )(ea, w0, b0, w1, b1)


def _tc_prep(x, cnt_a, cnt_b, gf, a0, ab0, a1, ab1, g0, gb0, g1, gb1,
             wg_e, bg_e, wg_n, bg_n):
    def body(x_ref, ca_ref, cb_ref, gf_ref, a0_ref, ab0_ref, a1_ref, ab1_ref,
             g0_ref, gb0_ref, g1_ref, gb1_ref, wge_ref, bge_ref, wgn_ref,
             bgn_ref, nl_ref, inv_ref, g_ref, egt_ref, ngt_ref):
        h = jnp.maximum(x_ref[...] @ a0_ref[...] + ab0_ref[...], 0.0)
        nl_ref[...] = _widen(h @ a1_ref[...] + ab1_ref[...], _NP)
        inv_ref[...] = 1.0 / jnp.maximum(ca_ref[...] + cb_ref[...], 1.0)
        hg = jnp.maximum(gf_ref[...] @ g0_ref[...] + gb0_ref[...], 0.0)
        g = hg @ g1_ref[...] + gb1_ref[...]
        g_ref[...] = g
        egt_ref[...] = g @ wge_ref[...] + bge_ref[...]
        ngt_ref[...] = g @ wgn_ref[...] + bgn_ref[...]

    return pl.pallas_call(
        body,
        out_shape=[
            jax.ShapeDtypeStruct((_NP, _W), jnp.float32),
            jax.ShapeDtypeStruct((_NP, _W), jnp.float32),
            jax.ShapeDtypeStruct((1, _L), jnp.float32),
            jax.ShapeDtypeStruct((1, _L), jnp.float32),
            jax.ShapeDtypeStruct((1, _L), jnp.float32),
        ],
    )(x, cnt_a, cnt_b, gf, a0, ab0, a1, ab1, g0, gb0, g1, gb1,
      wg_e, bg_e, wg_n, bg_n)


def _tc_edge(nlr, nlc, el, es, ws, bs, w0, egt, w1, b1):
    def body(nlr_ref, nlc_ref, el_ref, es_ref, ws_ref, bs_ref, w0_ref,
             egt_ref, w1_ref, b1_ref, out_ref):
        ein = (jnp.concatenate([el_ref[:, :_L], es_ref[...]], axis=1)
               @ ws_ref[...] + bs_ref[...])
        h = jnp.maximum(
            jnp.concatenate([nlr_ref[:, :_L], nlc_ref[:, :_L], ein], axis=1)
            @ w0_ref[...] + egt_ref[...], 0.0)
        out_ref[...] = _widen(h @ w1_ref[...] + b1_ref[...], _BE)

    ebw = pl.BlockSpec((_BE, _W), lambda i: (i, 0))
    ebl = pl.BlockSpec((_BE, _L), lambda i: (i, 0))
    return pl.pallas_call(
        body,
        grid=(_EP // _BE,),
        in_specs=[ebw, ebw, ebw, ebl,
                  _full((2 * _L, _L)), _full((1, _L)),
                  _full((3 * _L, _L)), _full((1, _L)),
                  _full((_L, _L)), _full((1, _L))],
        out_specs=ebw,
        out_shape=jax.ShapeDtypeStruct((_EP, _W), jnp.float32),
    )(nlr, nlc, el, es, ws, bs, w0, egt, w1, b1)


def _tc_node(nl, s_a, s_b, inv, v0, ngt, v1, b1, g0, gb0, g1, gb1, g,
             wg_e, bg_e, wg_n, bg_n):
    def body(nl_ref, sa_ref, sb_ref, inv_ref, v0_ref, ngt_ref, v1_ref,
             b1_ref, g0_ref, gb0_ref, g1_ref, gb1_ref, g_ref, wge_ref,
             bge_ref, wgn_ref, bgn_ref, ne_ref, gn_ref, egt_ref, ngt2_ref):
        s = sa_ref[:, :_L] + sb_ref[:, :_L]
        agg = s * inv_ref[:, :_L]
        h = jnp.maximum(
            jnp.concatenate([nl_ref[:, :_L], agg], axis=1) @ v0_ref[...]
            + ngt_ref[...], 0.0)
        ne = h @ v1_ref[...] + b1_ref[...]
        ne_ref[...] = _widen(ne, _NP)
        ridx = lax.broadcasted_iota(jnp.int32, (_NP, 1), 0)
        valid = (ridx < _N).astype(jnp.float32)
        n_g = jnp.sum(ne * valid, axis=0, keepdims=True) * (1.0 / _N)
        e_g = jnp.sum(s * valid, axis=0, keepdims=True) * (1.0 / _E)
        hg = jnp.maximum(
            jnp.concatenate([n_g, e_g, g_ref[...]], axis=1) @ g0_ref[...]
            + gb0_ref[...], 0.0)
        gn = hg @ g1_ref[...] + gb1_ref[...]
        gn_ref[...] = gn
        egt_ref[...] = gn @ wge_ref[...] + bge_ref[...]
        ngt2_ref[...] = gn @ wgn_ref[...] + bgn_ref[...]

    return pl.pallas_call(
        body,
        out_shape=[
            jax.ShapeDtypeStruct((_NP, _W), jnp.float32),
            jax.ShapeDtypeStruct((1, _L), jnp.float32),
            jax.ShapeDtypeStruct((1, _L), jnp.float32),
            jax.ShapeDtypeStruct((1, _L), jnp.float32),
        ],
    )(nl, s_a, s_b, inv, v0, ngt, v1, b1, g0, gb0, g1, gb1, g,
      wg_e, bg_e, wg_n, bg_n)


def _tc_diag(nl, d0, db0, d1, db1):
    def body(nl_ref, d0_ref, db0_ref, d1_ref, db1_ref, out_ref):
        h = jnp.maximum(nl_ref[:, :_L] @ d0_ref[...] + db0_ref[...], 0.0)
        dc = h @ d1_ref[...] + db1_ref[...]
        out_ref[...] = jnp.concatenate(
            [dc, jnp.zeros((_NP, _W - 2), jnp.float32)], axis=1)

    return pl.pallas_call(
        body,
        out_shape=jax.ShapeDtypeStruct((_NP, _W), jnp.float32),
    )(nl, d0, db0, d1, db1)


def _tc_final(el, dcr, row3, col3, e0, eb0, e1, eb1):
    nb = _EP // _BE

    def body(el_ref, dcr_ref, row_ref, col_ref, e0_ref, eb0_ref, e1_ref,
             eb1_ref, m_ref, l1_ref):
        i = pl.program_id(0)

        @pl.when(i == 0)
        def _():
            l1_ref[...] = jnp.zeros((1, 2), jnp.float32)

        h = jnp.maximum(el_ref[:, :_L] @ e0_ref[...] + eb0_ref[...], 0.0)
        ec = h @ e1_ref[...] + eb1_ref[...]
        r = row_ref[0]
        c = col_ref[0]
        diag = r == c
        dre = dcr_ref[:, 0:1]
        dim = dcr_ref[:, 1:2]
        m_re = jnp.where(diag, 1.0 + dre, ec[:, 0:1])
        m_im = jnp.where(diag, dim, ec[:, 1:2])
        m_ref[...] = jnp.concatenate([m_re, m_im], axis=1)
        eidx = i * _BE + lax.broadcasted_iota(jnp.int32, (_BE, 1), 0)
        offv = jnp.logical_and(jnp.logical_not(diag), eidx < _E)
        sq = m_re * m_re + m_im * m_im
        mag = jnp.sqrt(jnp.where(offv, sq, 1.0))
        bsum = jnp.sum(jnp.where(offv, mag, 0.0))
        bcnt = jnp.sum(offv.astype(jnp.float32))
        upd = jnp.concatenate(
            [jnp.full((1, 1), bsum), jnp.full((1, 1), bcnt)], axis=1)
        l1_ref[...] = l1_ref[...] + upd

        @pl.when(i == nb - 1)
        def _():
            v = l1_ref[...]
            l1 = v[:, 0:1] / jnp.maximum(v[:, 1:2], 1.0)
            l1_ref[...] = jnp.concatenate([l1, v[:, 1:2]], axis=1)

    ebw = pl.BlockSpec((_BE, _W), lambda i: (i, 0))
    ib = pl.BlockSpec((1, _BE, 1), lambda i: (i, 0, 0))
    return pl.pallas_call(
        body,
        grid=(nb,),
        in_specs=[ebw, ebw, ib, ib,
                  _full((_L, _L)), _full((1, _L)),
                  _full((_L, 2)), _full((1, 2))],
        out_specs=[pl.BlockSpec((_BE, 2), lambda i: (i, 0)),
                   pl.BlockSpec((1, 2), lambda i: (0, 0))],
        out_shape=[jax.ShapeDtypeStruct((_EP, 2), jnp.float32),
                   jax.ShapeDtypeStruct((1, 2), jnp.float32)],
    )(el, dcr, row3, col3, e0, eb0, e1, eb1)


def kernel(x, edge_attr, global_features, params, edge_index):
    f32 = jnp.float32
    row = edge_index[0].astype(jnp.int32)
    col = edge_index[1].astype(jnp.int32)
    pad_e = _EP - _E

    row_g = jnp.concatenate([row, jnp.zeros((pad_e,), jnp.int32)])
    col_g = jnp.concatenate([col, jnp.zeros((pad_e,), jnp.int32)])
    idx_g2 = jnp.stack([row_g, col_g]).reshape(2, _NW, _CPW, _CHUNK)
    idx_g1 = row_g.reshape(1, _NW, _CPW, _CHUNK)
    idx_s = jnp.concatenate(
        [row, jnp.full((pad_e,), _N, jnp.int32)]).reshape(
            _NW, _CPW, _CHUNK)
    row3 = row_g.reshape(_EP // _BE, _BE, 1)
    col3 = col_g.reshape(_EP // _BE, _BE, 1)

    zrows = jnp.zeros((_RPT, _W), f32)
    ones_d = jnp.ones((_EP, _W), f32)
    x_pad = jnp.concatenate([x, jnp.zeros((_NP - _N, 9), f32)])
    ea_pad = jnp.concatenate([edge_attr, jnp.zeros((pad_e, 8), f32)])
    gf = global_features.reshape(1, 4)

    p = params

    def wb(mlp, i):
        return mlp[i]["W"], mlp[i]["b"].reshape(1, -1)

    cnt = _sc_scatter_add(ones_d, idx_s, zrows)

    b0w, b0b = wb(p["edge_enc"], 0)
    b1w, b1b = wb(p["edge_enc"], 1)
    es = _tc_edge_enc(ea_pad, b0w, b0b, b1w, b1b)

    a0w, a0b = wb(p["node_enc"], 0)
    a1w, a1b = wb(p["node_enc"], 1)
    g0w, g0b = wb(p["global_enc"], 0)
    g1w, g1b = wb(p["global_enc"], 1)
    e0 = p["proc"][0]
    nl, inv, g, egt, ngt = _tc_prep(
        x_pad, cnt[0], cnt[1], gf, a0w, a0b, a1w, a1b, g0w, g0b, g1w, g1b,
        e0["edge"][0]["W"][0:_L], e0["edge"][0]["b"].reshape(1, -1),
        e0["node"][0]["W"][0:_L], e0["node"][0]["b"].reshape(1, -1))

    el = _widen(es, _EP)
    for i in range(_STEPS):
        blk = p["proc"][i]
        sp = p["skip"][i]
        nlrc = _sc_gather(nl, idx_g2)
        ew0 = blk["edge"][0]["W"][_L:4 * _L]
        ew1, eb1 = wb(blk["edge"], 1)
        e_emb = _tc_edge(nlrc[:_EP], nlrc[_EP:], el, es,
                         sp["W"], sp["b"].reshape(1, -1), ew0, egt, ew1, eb1)
        sums = _sc_scatter_add(e_emb, idx_s, zrows)
        nxt = p["proc"][(i + 1) % _STEPS]
        nw0 = blk["node"][0]["W"][_L:3 * _L]
        nw1, nb1 = wb(blk["node"], 1)
        gw0, gb0 = wb(blk["global"], 0)
        gw1, gb1 = wb(blk["global"], 1)
        nl, g, egt, ngt = _tc_node(
            nl, sums[0], sums[1], inv, nw0, ngt, nw1, nb1,
            gw0, gb0, gw1, gb1, g,
            nxt["edge"][0]["W"][0:_L], nxt["edge"][0]["b"].reshape(1, -1),
            nxt["node"][0]["W"][0:_L], nxt["node"][0]["b"].reshape(1, -1))
        el = e_emb

    d0w, d0b = wb(p["diag_dec"], 0)
    d1w, d1b = wb(p["diag_dec"], 1)
    dpad = _tc_diag(nl, d0w, d0b, d1w, d1b)
    dcr = _sc_gather(dpad, idx_g1)

    ed0w, ed0b = wb(p["edge_dec"], 0)
    ed1w, ed1b = wb(p["edge_dec"], 1)
    m, l1v = _tc_final(el, dcr, row3, col3, ed0w, ed0b, ed1w, ed1b)
    return m[:_E], l1v[0, 0]


# speedup vs baseline: 1.7900x; 1.0419x over previous
import functools

import jax
import jax.numpy as jnp
from jax import lax
from jax.experimental import pallas as pl
from jax.experimental.pallas import tpu as pltpu
from jax.experimental.pallas import tpu_sc as plsc

_N = 10000
_E = 160000
_L = 64
_W = 128
_STEPS = 6
_NW = 32
_CHUNK = 128
_CPW = 40
_EPW = _CHUNK * _CPW
_EP = _NW * _EPW
_NP = 10112
_RPT = _NP // 16
_BE = 2048


def _mesh():
    return plsc.VectorSubcoreMesh(core_axis_name="c", subcore_axis_name="s")


def _widen(v, rows):
    return jnp.concatenate([v, jnp.zeros((rows, _W - v.shape[1]), jnp.float32)],
                           axis=1)


def _split_bf16(v):
    hi = v.astype(jnp.bfloat16)
    lo = (v - hi.astype(jnp.float32)).astype(jnp.bfloat16)
    return jnp.concatenate([hi, lo], axis=1)


def _join_bf16(hl):
    return hl[:, :_L].astype(jnp.float32) + hl[:, _L:].astype(jnp.float32)


_NB = 4
_NG = 2


def _sc_gather(table, idx):
    s_sets = idx.shape[0]
    t_total = s_sets * _CPW

    @functools.partial(
        pl.kernel,
        out_type=jax.ShapeDtypeStruct((s_sets * _EP, _L), jnp.float32),
        mesh=_mesh(),
        scratch_types=(
            [pltpu.VMEM((s_sets * _CPW, _CHUNK), jnp.int32)]
            + [pltpu.VMEM((_CHUNK, _L), jnp.float32)] * _NG
            + [pltpu.SemaphoreType.DMA] * (2 * _NG)
            + [pltpu.VMEM_SHARED((_NP, _L), jnp.float32)]
        ),
    )
    def k(table_hbm, idx_hbm, out_hbm, idx_v, *rest):
        bufs = rest[:_NG]
        gs = rest[_NG:2 * _NG]
        os_ = rest[2 * _NG:3 * _NG]
        tbl = rest[3 * _NG]
        cid = lax.axis_index("c")
        sid = lax.axis_index("s")
        w = cid * 16 + sid
        base = w * _EPW
        pltpu.sync_copy(table_hbm.at[pl.ds(sid * _RPT, _RPT)],
                        tbl.at[pl.ds(sid * _RPT, _RPT)])
        for a in range(s_sets):
            pltpu.sync_copy(idx_hbm.at[a, w],
                            idx_v.at[pl.ds(a * _CPW, _CPW)])
        plsc.subcore_barrier()

        def out_off(t):
            if s_sets == 1:
                return base + t * _CHUNK
            a = (t >= _CPW).astype(jnp.int32)
            return a * _EP + base + (t - a * _CPW) * _CHUNK

        def start_gather(t, buf, sem):
            pltpu.async_copy(tbl.at[idx_v.at[t]], buf, sem)

        def wait_gather(buf, sem):
            pltpu.make_async_copy(tbl.at[idx_v.at[0]], buf, sem).wait()

        def start_out(t, buf, sem):
            pltpu.async_copy(buf, out_hbm.at[pl.ds(out_off(t), _CHUNK)],
                             sem)

        def wait_out(buf, sem):
            pltpu.make_async_copy(buf, out_hbm.at[pl.ds(0, _CHUNK)],
                                  sem).wait()

        for b in range(_NG):
            start_gather(b, bufs[b], gs[b])

        half = _NG // 2

        @pl.loop(0, t_total // _NG)
        def _(u):
            tb = _NG * u
            for b in range(_NG):
                t = tb + b
                wait_gather(bufs[b], gs[b])
                start_out(t, bufs[b], os_[b])
                bp = (b + half) % _NG

                @pl.when(jnp.logical_and(t >= half, t + half < t_total))
                def _():
                    wait_out(bufs[bp], os_[bp])
                    start_gather(t + half, bufs[bp], gs[bp])

        for b in range(_NG):
            wait_out(bufs[b], os_[b])

    return k(table, idx)


def _sc_scatter_add(data, idx, zrows):
    nbs = 2

    @functools.partial(
        pl.kernel,
        out_type=jax.ShapeDtypeStruct((2, _NP, _L), jnp.float32),
        mesh=_mesh(),
        scratch_types=(
            [pltpu.VMEM((_CPW, _CHUNK), jnp.int32)]
            + [pltpu.VMEM((_CHUNK, _L), jnp.float32)] * nbs
            + [pltpu.SemaphoreType.DMA] * (2 * nbs)
            + [pltpu.VMEM_SHARED((_NP, _L), jnp.float32)]
        ),
    )
    def k(data_hbm, idx_hbm, z_hbm, out_hbm, idx_v, *rest):
        bufs = rest[:nbs]
        ls = rest[nbs:2 * nbs]
        ss = rest[2 * nbs:3 * nbs]
        acc = rest[3 * nbs]
        cid = lax.axis_index("c")
        sid = lax.axis_index("s")
        w = cid * 16 + sid
        base = w * _EPW
        pltpu.sync_copy(z_hbm, acc.at[pl.ds(sid * _RPT, _RPT)])
        pltpu.sync_copy(idx_hbm.at[w], idx_v)
        plsc.subcore_barrier()

        def start_load(t, buf, sem):
            pltpu.async_copy(data_hbm.at[pl.ds(base + t * _CHUNK, _CHUNK)],
                             buf, sem)

        def wait_load(buf, sem):
            pltpu.make_async_copy(data_hbm.at[pl.ds(0, _CHUNK)], buf,
                                  sem).wait()

        def start_scat(t, buf, sem):
            pltpu.async_copy(buf, acc.at[idx_v.at[t]], sem, add=True)

        def wait_scat(buf, sem):
            pltpu.make_async_copy(buf, acc.at[idx_v.at[0]], sem).wait()

        for b in range(nbs):
            start_load(b, bufs[b], ls[b])

        @pl.loop(0, _CPW // nbs)
        def _(u):
            tb = nbs * u
            for b in range(nbs):
                t = tb + b
                wait_load(bufs[b], ls[b])
                start_scat(t, bufs[b], ss[b])

                @pl.when(t + nbs < _CPW)
                def _():
                    wait_scat(bufs[b], ss[b])
                    start_load(t + nbs, bufs[b], ls[b])

        for b in range(nbs):
            wait_scat(bufs[b], ss[b])
        plsc.subcore_barrier()
        pltpu.sync_copy(acc.at[pl.ds(sid * _RPT, _RPT)],
                        out_hbm.at[cid, pl.ds(sid * _RPT, _RPT)])

    return k(data, idx, zrows)


def _full(shape):
    return pl.BlockSpec(shape, lambda i: tuple(0 for _ in shape))


def _tc_edge_enc(ea, w0, b0, w1, b1):
    def body(ea_ref, w0_ref, b0_ref, w1_ref, b1_ref, out_ref):
        h = jnp.maximum(ea_ref[...] @ w0_ref[...] + b0_ref[...], 0.0)
        out_ref[...] = h @ w1_ref[...] + b1_ref[...]

    return pl.pallas_call(
        body,
        grid=(_EP // _BE,),
        in_specs=[
            pl.BlockSpec((_BE, 8), lambda i: (i, 0)),
            _full((8, _L)), _full((1, _L)), _full((_L, _L)), _full((1, _L)),
        ],
        out_specs=pl.BlockSpec((_BE, _L), lambda i: (i, 0)),
        out_shape=jax.ShapeDtypeStruct((_EP, _L), jnp.float32),
    )(ea, w0, b0, w1, b1)


def _tc_prep(x, cnt_a, cnt_b, gf, a0, ab0, a1, ab1, g0, gb0, g1, gb1,
             wg_e, bg_e, wg_n, bg_n):
    def body(x_ref, ca_ref, cb_ref, gf_ref, a0_ref, ab0_ref, a1_ref, ab1_ref,
             g0_ref, gb0_ref, g1_ref, gb1_ref, wge_ref, bge_ref, wgn_ref,
             bgn_ref, nl_ref, inv_ref, g_ref, egt_ref, ngt_ref):
        h = jnp.maximum(x_ref[...] @ a0_ref[...] + ab0_ref[...], 0.0)
        nl_ref[...] = h @ a1_ref[...] + ab1_ref[...]
        inv_ref[...] = 1.0 / jnp.maximum(ca_ref[...] + cb_ref[...], 1.0)
        hg = jnp.maximum(gf_ref[...] @ g0_ref[...] + gb0_ref[...], 0.0)
        g = hg @ g1_ref[...] + gb1_ref[...]
        g_ref[...] = g
        egt_ref[...] = g @ wge_ref[...] + bge_ref[...]
        ngt_ref[...] = g @ wgn_ref[...] + bgn_ref[...]

    return pl.pallas_call(
        body,
        out_shape=[
            jax.ShapeDtypeStruct((_NP, _L), jnp.float32),
            jax.ShapeDtypeStruct((_NP, _L), jnp.float32),
            jax.ShapeDtypeStruct((1, _L), jnp.float32),
            jax.ShapeDtypeStruct((1, _L), jnp.float32),
            jax.ShapeDtypeStruct((1, _L), jnp.float32),
        ],
    )(x, cnt_a, cnt_b, gf, a0, ab0, a1, ab1, g0, gb0, g1, gb1,
      wg_e, bg_e, wg_n, bg_n)


def _tc_edge(nlr, nlc, el, es, ws, bs, w0, egt, w1, b1):
    def body(nlr_ref, nlc_ref, el_ref, es_ref, ws_ref, bs_ref, w0_ref,
             egt_ref, w1_ref, b1_ref, out_ref):
        ein = (jnp.concatenate([el_ref[...], es_ref[...]], axis=1)
               @ ws_ref[...] + bs_ref[...])
        h = jnp.maximum(
            jnp.concatenate([nlr_ref[...], nlc_ref[...], ein], axis=1)
            @ w0_ref[...] + egt_ref[...], 0.0)
        out_ref[...] = h @ w1_ref[...] + b1_ref[...]

    ebl = pl.BlockSpec((_BE, _L), lambda i: (i, 0))
    return pl.pallas_call(
        body,
        grid=(_EP // _BE,),
        in_specs=[ebl, ebl, ebl, ebl,
                  _full((2 * _L, _L)), _full((1, _L)),
                  _full((3 * _L, _L)), _full((1, _L)),
                  _full((_L, _L)), _full((1, _L))],
        out_specs=ebl,
        out_shape=jax.ShapeDtypeStruct((_EP, _L), jnp.float32),
    )(nlr, nlc, el, es, ws, bs, w0, egt, w1, b1)


def _tc_node(nl, s_a, s_b, inv, v0, ngt, v1, b1, g0, gb0, g1, gb1, g,
             wg_e, bg_e, wg_n, bg_n):
    def body(nl_ref, sa_ref, sb_ref, inv_ref, v0_ref, ngt_ref, v1_ref,
             b1_ref, g0_ref, gb0_ref, g1_ref, gb1_ref, g_ref, wge_ref,
             bge_ref, wgn_ref, bgn_ref, ne_ref, gn_ref, egt_ref, ngt2_ref):
        s = sa_ref[...] + sb_ref[...]
        agg = s * inv_ref[...]
        h = jnp.maximum(
            jnp.concatenate([nl_ref[...], agg], axis=1) @ v0_ref[...]
            + ngt_ref[...], 0.0)
        ne = h @ v1_ref[...] + b1_ref[...]
        ne_ref[...] = ne
        ridx = lax.broadcasted_iota(jnp.int32, (_NP, 1), 0)
        valid = (ridx < _N).astype(jnp.float32)
        n_g = jnp.sum(ne * valid, axis=0, keepdims=True) * (1.0 / _N)
        e_g = jnp.sum(s * valid, axis=0, keepdims=True) * (1.0 / _E)
        hg = jnp.maximum(
            jnp.concatenate([n_g, e_g, g_ref[...]], axis=1) @ g0_ref[...]
            + gb0_ref[...], 0.0)
        gn = hg @ g1_ref[...] + gb1_ref[...]
        gn_ref[...] = gn
        egt_ref[...] = gn @ wge_ref[...] + bge_ref[...]
        ngt2_ref[...] = gn @ wgn_ref[...] + bgn_ref[...]

    return pl.pallas_call(
        body,
        out_shape=[
            jax.ShapeDtypeStruct((_NP, _L), jnp.float32),
            jax.ShapeDtypeStruct((1, _L), jnp.float32),
            jax.ShapeDtypeStruct((1, _L), jnp.float32),
            jax.ShapeDtypeStruct((1, _L), jnp.float32),
        ],
    )(nl, s_a, s_b, inv, v0, ngt, v1, b1, g0, gb0, g1, gb1, g,
      wg_e, bg_e, wg_n, bg_n)


def _tc_diag(nl, d0, db0, d1, db1):
    def body(nl_ref, d0_ref, db0_ref, d1_ref, db1_ref, out_ref):
        h = jnp.maximum(nl_ref[...] @ d0_ref[...] + db0_ref[...], 0.0)
        dc = h @ d1_ref[...] + db1_ref[...]
        out_ref[...] = jnp.concatenate(
            [dc, jnp.zeros((_NP, _L - 2), jnp.float32)], axis=1)

    return pl.pallas_call(
        body,
        out_shape=jax.ShapeDtypeStruct((_NP, _L), jnp.float32),
    )(nl, d0, db0, d1, db1)


def _tc_final(el, dcr, row3, col3, e0, eb0, e1, eb1):
    nb = _EP // _BE

    def body(el_ref, dcr_ref, row_ref, col_ref, e0_ref, eb0_ref, e1_ref,
             eb1_ref, m_ref, l1_ref):
        i = pl.program_id(0)

        @pl.when(i == 0)
        def _():
            l1_ref[...] = jnp.zeros((1, 2), jnp.float32)

        h = jnp.maximum(el_ref[...] @ e0_ref[...] + eb0_ref[...], 0.0)
        ec = h @ e1_ref[...] + eb1_ref[...]
        r = row_ref[0]
        c = col_ref[0]
        diag = r == c
        dre = dcr_ref[:, 0:1]
        dim = dcr_ref[:, 1:2]
        m_re = jnp.where(diag, 1.0 + dre, ec[:, 0:1])
        m_im = jnp.where(diag, dim, ec[:, 1:2])
        m_ref[...] = jnp.concatenate([m_re, m_im], axis=1)
        eidx = i * _BE + lax.broadcasted_iota(jnp.int32, (_BE, 1), 0)
        offv = jnp.logical_and(jnp.logical_not(diag), eidx < _E)
        sq = m_re * m_re + m_im * m_im
        mag = jnp.sqrt(jnp.where(offv, sq, 1.0))
        bsum = jnp.sum(jnp.where(offv, mag, 0.0))
        bcnt = jnp.sum(offv.astype(jnp.float32))
        upd = jnp.concatenate(
            [jnp.full((1, 1), bsum), jnp.full((1, 1), bcnt)], axis=1)
        l1_ref[...] = l1_ref[...] + upd

        @pl.when(i == nb - 1)
        def _():
            v = l1_ref[...]
            l1 = v[:, 0:1] / jnp.maximum(v[:, 1:2], 1.0)
            l1_ref[...] = jnp.concatenate([l1, v[:, 1:2]], axis=1)

    ebl = pl.BlockSpec((_BE, _L), lambda i: (i, 0))
    ib = pl.BlockSpec((1, _BE, 1), lambda i: (i, 0, 0))
    return pl.pallas_call(
        body,
        grid=(nb,),
        in_specs=[ebl, ebl, ib, ib,
                  _full((_L, _L)), _full((1, _L)),
                  _full((_L, 2)), _full((1, 2))],
        out_specs=[pl.BlockSpec((_BE, 2), lambda i: (i, 0)),
                   pl.BlockSpec((1, 2), lambda i: (0, 0))],
        out_shape=[jax.ShapeDtypeStruct((_EP, 2), jnp.float32),
                   jax.ShapeDtypeStruct((1, 2), jnp.float32)],
    )(el, dcr, row3, col3, e0, eb0, e1, eb1)


def kernel(x, edge_attr, global_features, params, edge_index):
    f32 = jnp.float32
    row = edge_index[0].astype(jnp.int32)
    col = edge_index[1].astype(jnp.int32)
    pad_e = _EP - _E

    row_g = jnp.concatenate([row, jnp.zeros((pad_e,), jnp.int32)])
    col_g = jnp.concatenate([col, jnp.zeros((pad_e,), jnp.int32)])
    idx_g2 = jnp.stack([row_g, col_g]).reshape(2, _NW, _CPW, _CHUNK)
    idx_g1 = row_g.reshape(1, _NW, _CPW, _CHUNK)
    idx_s = jnp.concatenate(
        [row, jnp.full((pad_e,), _N, jnp.int32)]).reshape(
            _NW, _CPW, _CHUNK)
    row3 = row_g.reshape(_EP // _BE, _BE, 1)
    col3 = col_g.reshape(_EP // _BE, _BE, 1)

    zrows = jnp.zeros((_RPT, _L), f32)
    ones_d = jnp.ones((_EP, _L), f32)
    x_pad = jnp.concatenate([x, jnp.zeros((_NP - _N, 9), f32)])
    ea_pad = jnp.concatenate([edge_attr, jnp.zeros((pad_e, 8), f32)])
    gf = global_features.reshape(1, 4)

    p = params

    def wb(mlp, i):
        return mlp[i]["W"], mlp[i]["b"].reshape(1, -1)

    cnt = _sc_scatter_add(ones_d, idx_s, zrows)

    b0w, b0b = wb(p["edge_enc"], 0)
    b1w, b1b = wb(p["edge_enc"], 1)
    es = _tc_edge_enc(ea_pad, b0w, b0b, b1w, b1b)

    a0w, a0b = wb(p["node_enc"], 0)
    a1w, a1b = wb(p["node_enc"], 1)
    g0w, g0b = wb(p["global_enc"], 0)
    g1w, g1b = wb(p["global_enc"], 1)
    e0 = p["proc"][0]
    nl, inv, g, egt, ngt = _tc_prep(
        x_pad, cnt[0], cnt[1], gf, a0w, a0b, a1w, a1b, g0w, g0b, g1w, g1b,
        e0["edge"][0]["W"][0:_L], e0["edge"][0]["b"].reshape(1, -1),
        e0["node"][0]["W"][0:_L], e0["node"][0]["b"].reshape(1, -1))

    el = es
    for i in range(_STEPS):
        blk = p["proc"][i]
        sp = p["skip"][i]
        nlrc = _sc_gather(nl, idx_g2)
        ew0 = blk["edge"][0]["W"][_L:4 * _L]
        ew1, eb1 = wb(blk["edge"], 1)
        e_emb = _tc_edge(nlrc[:_EP], nlrc[_EP:], el, es,
                         sp["W"], sp["b"].reshape(1, -1), ew0, egt, ew1, eb1)
        sums = _sc_scatter_add(e_emb, idx_s, zrows)
        nxt = p["proc"][(i + 1) % _STEPS]
        nw0 = blk["node"][0]["W"][_L:3 * _L]
        nw1, nb1 = wb(blk["node"], 1)
        gw0, gb0 = wb(blk["global"], 0)
        gw1, gb1 = wb(blk["global"], 1)
        nl, g, egt, ngt = _tc_node(
            nl, sums[0], sums[1], inv, nw0, ngt, nw1, nb1,
            gw0, gb0, gw1, gb1, g,
            nxt["edge"][0]["W"][0:_L], nxt["edge"][0]["b"].reshape(1, -1),
            nxt["node"][0]["W"][0:_L], nxt["node"][0]["b"].reshape(1, -1))
        el = e_emb

    d0w, d0b = wb(p["diag_dec"], 0)
    d1w, d1b = wb(p["diag_dec"], 1)
    dpad = _tc_diag(nl, d0w, d0b, d1w, d1b)
    dcr = _sc_gather(dpad, idx_g1)

    ed0w, ed0b = wb(p["edge_dec"], 0)
    ed1w, ed1b = wb(p["edge_dec"], 1)
    m, l1v = _tc_final(el, dcr, row3, col3, ed0w, ed0b, ed1w, ed1b)
    return m[:_E], l1v[0, 0]
